# baseline (speedup 1.0000x reference)
import functools

import jax
import jax.numpy as jnp
from jax import lax
from jax.experimental import pallas as pl
from jax.experimental.pallas import tpu as pltpu
from jax.experimental.pallas import tpu_sc as plsc

B, V, K, R, A = 1024, 8, 16, 8192, 6
N_ANG = 9
C = V * K
L = 16
NTILES = 32
BPT = B // NTILES
RS = 3072
RT = R - RS
TT = 512
NCHUNK = RS // L


def _sc_rule_kernel(x_rep, c_flat, w_flat, idx_t, or_t, oc_pad):
    mesh = plsc.VectorSubcoreMesh(core_axis_name="c", subcore_axis_name="s")

    @functools.partial(
        pl.kernel,
        mesh=mesh,
        compiler_params=pltpu.CompilerParams(needs_layout_passes=False),
        out_type=jax.ShapeDtypeStruct((B * 3 * L,), jnp.float32),
        scratch_types=[
            pltpu.VMEM((BPT * C,), jnp.float32),
            pltpu.VMEM((BPT // 2 * C,), jnp.int32),
            pltpu.VMEM((C,), jnp.float32),
            pltpu.VMEM((C,), jnp.float32),
            pltpu.VMEM((RS * A,), jnp.int32),
            pltpu.VMEM((RS * 2,), jnp.int32),
            pltpu.VMEM((C,), jnp.float32),
            pltpu.VMEM((BPT * 3 * L,), jnp.float32),
        ],
    )
    def k(x_hbm, c_hbm, w_hbm, idx_hbm, or_hbm, oc_hbm, out_hbm,
          xs, fz, cv, nv, idxv, orv, ocv, acc):
        wid = lax.axis_index("s") * 2 + lax.axis_index("c")

        pltpu.sync_copy(x_hbm.at[pl.ds(wid * BPT * C, BPT * C)], xs)
        pltpu.sync_copy(c_hbm, cv)
        pltpu.sync_copy(w_hbm, nv)
        pltpu.sync_copy(idx_hbm, idxv)
        pltpu.sync_copy(or_hbm, orv)
        pltpu.sync_copy(oc_hbm, ocv)

        for t in range(C // L):
            wv = nv[pl.ds(t * L, L)]
            nv[pl.ds(t * L, L)] = -0.5 / (wv * wv)

        zero = jnp.zeros((L,), jnp.float32)

        def fuzz_body(bp, carry):
            for t in range(C // L):
                col = pl.ds(t * L, L)
                cvt = cv[col]
                nvt = nv[col]
                xe = xs[pl.ds((2 * bp) * C + t * L, L)]
                xo = xs[pl.ds((2 * bp + 1) * C + t * L, L)]
                de = xe - cvt
                do = xo - cvt
                fe = jnp.exp(de * de * nvt)
                fo = jnp.exp(do * do * nvt)
                packed = plsc.pack(fe, fo, format=plsc.PackFormat.INTERLEAVED)
                fz[pl.ds(bp * C + t * L, L)] = plsc.bitcast(packed, jnp.int32)
            for j in range(6):
                acc[pl.ds(bp * 6 * L + j * L, L)] = zero
            return carry

        lax.fori_loop(0, BPT // 2, fuzz_body, 0)

        riota = lax.iota(jnp.int32, L)

        @plsc.parallel_loop(0, NCHUNK, unroll=4)
        def chunk_body(i):
            cvec = riota * A + i * (L * A)
            ia = [plsc.load_gather(idxv, [cvec + a]) for a in range(A)]
            ovec = riota * 2 + i * (L * 2)
            ow0 = plsc.load_gather(ocv, [plsc.load_gather(orv, [ovec])])
            ow1 = plsc.load_gather(ocv, [plsc.load_gather(orv, [ovec + 1])])
            for bp in range(BPT // 2):
                boff = bp * C
                g = [
                    plsc.bitcast(
                        plsc.load_gather(fz, [ia[a] + boff]), jnp.bfloat16
                    )
                    for a in range(A)
                ]
                w01 = jnp.minimum(g[0], g[1])
                w23 = jnp.minimum(g[2], g[3])
                w45 = jnp.minimum(g[4], g[5])
                w = jnp.minimum(jnp.minimum(w01, w23), w45)
                we, wo = plsc.unpack(w, format=plsc.PackFormat.INTERLEAVED)
                o = bp * 6 * L
                plsc.addupdate(acc.at[pl.ds(o, L)], we * ow0)
                plsc.addupdate(acc.at[pl.ds(o + L, L)], we * ow1)
                plsc.addupdate(acc.at[pl.ds(o + 2 * L, L)], we)
                plsc.addupdate(acc.at[pl.ds(o + 3 * L, L)], wo * ow0)
                plsc.addupdate(acc.at[pl.ds(o + 4 * L, L)], wo * ow1)
                plsc.addupdate(acc.at[pl.ds(o + 5 * L, L)], wo)

        pltpu.sync_copy(acc, out_hbm.at[pl.ds(wid * BPT * 3 * L, BPT * 3 * L)])

    return k(x_rep, c_flat, w_flat, idx_t, or_t, oc_pad)


def _tc_rule_kernel(x_rep2, c_flat, w_flat, idx_blk, or_blk, oc_mat):
    nblk = RT // TT

    def body(xr_ref, cv_ref, wv_ref, idx_ref, or_ref, ocm_ref, out_ref,
             fzs, accs):
        step = pl.program_id(0)

        @pl.when(step == 0)
        def _init():
            d = xr_ref[...] - cv_ref[...]
            wv = wv_ref[...]
            fzs[...] = jnp.exp(-(d * d) / (2.0 * wv * wv)).astype(jnp.bfloat16)
            accs[...] = jnp.zeros((B, 3), jnp.float32)

        fzb = fzs[...]
        iota = lax.broadcasted_iota(jnp.int32, (C, TT), 0)
        w = None
        for a in range(A):
            oh = (iota == idx_ref[0, a, :][None, :]).astype(jnp.bfloat16)
            ga = jnp.dot(fzb, oh, preferred_element_type=jnp.float32)
            w = ga if w is None else jnp.minimum(w, ga)
        i16 = lax.broadcasted_iota(jnp.int32, (TT, L), 1)
        ohc = jnp.logical_or(
            i16 == or_ref[0, 0, :][:, None], i16 == or_ref[0, 1, :][:, None]
        ).astype(jnp.float32)
        ow3 = jnp.dot(ohc, ocm_ref[...], preferred_element_type=jnp.float32,
                      precision=jax.lax.Precision.HIGHEST)
        part = jnp.dot(w.astype(jnp.bfloat16), ow3.astype(jnp.bfloat16),
                       preferred_element_type=jnp.float32)
        accs[...] += part

        @pl.when(step == nblk - 1)
        def _fini():
            out_ref[...] = accs[...]

    return pl.pallas_call(
        body,
        grid=(nblk,),
        in_specs=[
            pl.BlockSpec((B, C), lambda j: (0, 0)),
            pl.BlockSpec((1, C), lambda j: (0, 0)),
            pl.BlockSpec((1, C), lambda j: (0, 0)),
            pl.BlockSpec((1, A, TT), lambda j: (j, 0, 0)),
            pl.BlockSpec((1, 2, TT), lambda j: (j, 0, 0)),
            pl.BlockSpec((L, 3), lambda j: (0, 0)),
        ],
        out_specs=pl.BlockSpec((B, 3), lambda j: (0, 0)),
        out_shape=jax.ShapeDtypeStruct((B, 3), jnp.float32),
        scratch_shapes=[
            pltpu.VMEM((B, C), jnp.bfloat16),
            pltpu.VMEM((B, 3), jnp.float32),
        ],
    )(x_rep2, c_flat.reshape(1, C), w_flat.reshape(1, C), idx_blk, or_blk,
      oc_mat)


def _tc_epilogue(acc_flat, tc3, out_scaling, out_bias):
    def body(a_ref, t_ref, s_ref, b_ref, o_ref):
        a = a_ref[...]
        t3 = t_ref[...]
        s0 = jnp.sum(a[:, 0:L], axis=1, keepdims=True) + t3[:, 0:1]
        s1 = jnp.sum(a[:, L:2 * L], axis=1, keepdims=True) + t3[:, 1:2]
        sd = jnp.sum(a[:, 2 * L:3 * L], axis=1, keepdims=True) + t3[:, 2:3]
        denom = jnp.maximum(sd, 1e-12)
        z = jnp.concatenate([s0, s1], axis=1) / denom
        o_ref[...] = jnp.tanh(z) * s_ref[...] + b_ref[...]

    return pl.pallas_call(
        body,
        out_shape=jax.ShapeDtypeStruct((B, 2), jnp.float32),
    )(acc_flat, tc3, out_scaling, out_bias)


def kernel(x, in_centers, in_widths, out_centers, out_scaling, out_bias,
           input_rules, output_rules):
    x_rep2 = jnp.repeat(x, K, axis=1)
    x_rep = x_rep2.reshape(B * C)
    c_flat = in_centers.reshape(C)
    w_flat = in_widths.reshape(C)
    idx_sc = input_rules[:RS].reshape(RS * A)
    or_sc = output_rules[:RS].reshape(RS * 2)
    oc_pad = jnp.pad(out_centers, (0, C - out_centers.shape[0]))

    idx_blk = (input_rules[RS:].reshape(RT // TT, TT, A)
               .transpose(0, 2, 1))
    or_blk = (output_rules[RS:].reshape(RT // TT, TT, 2)
              .transpose(0, 2, 1))
    n12 = out_centers.shape[0]
    ocp = jnp.pad(out_centers, (0, L - n12))
    k16 = jnp.arange(L)
    oc_mat = jnp.stack(
        [jnp.where(k16 < N_ANG, ocp, 0.0),
         jnp.where(k16 >= N_ANG, ocp, 0.0),
         (k16 < N_ANG).astype(jnp.float32)], axis=1)

    acc = _sc_rule_kernel(x_rep, c_flat, w_flat, idx_sc, or_sc, oc_pad)
    tc3 = _tc_rule_kernel(x_rep2, c_flat, w_flat, idx_blk, or_blk, oc_mat)
    return _tc_epilogue(acc.reshape(B, 3 * L), tc3, out_scaling, out_bias)


# speedup vs baseline: 1.0483x; 1.0483x over previous
import functools

import jax
import jax.numpy as jnp
from jax import lax
from jax.experimental import pallas as pl
from jax.experimental.pallas import tpu as pltpu
from jax.experimental.pallas import tpu_sc as plsc

B, V, K, R, A = 1024, 8, 16, 8192, 6
N_ANG = 9
C = V * K
L = 16
NTILES = 32
BPT = B // NTILES
RS = 2560
RT = R - RS
TT = 512
NCHUNK = RS // L


def _sc_rule_kernel(x_rep, c_flat, w_flat, idx_t, or_t, oc_pad):
    mesh = plsc.VectorSubcoreMesh(core_axis_name="c", subcore_axis_name="s")

    @functools.partial(
        pl.kernel,
        mesh=mesh,
        compiler_params=pltpu.CompilerParams(needs_layout_passes=False),
        out_type=jax.ShapeDtypeStruct((B * 3 * L,), jnp.float32),
        scratch_types=[
            pltpu.VMEM((BPT * C,), jnp.float32),
            pltpu.VMEM((BPT // 2 * C,), jnp.int32),
            pltpu.VMEM((C,), jnp.float32),
            pltpu.VMEM((C,), jnp.float32),
            pltpu.VMEM((RS * A,), jnp.int32),
            pltpu.VMEM((RS * 2,), jnp.int32),
            pltpu.VMEM((C,), jnp.float32),
            pltpu.VMEM((BPT * 3 * L,), jnp.float32),
        ],
    )
    def k(x_hbm, c_hbm, w_hbm, idx_hbm, or_hbm, oc_hbm, out_hbm,
          xs, fz, cv, nv, idxv, orv, ocv, acc):
        wid = lax.axis_index("s") * 2 + lax.axis_index("c")

        pltpu.sync_copy(x_hbm.at[pl.ds(wid * BPT * C, BPT * C)], xs)
        pltpu.sync_copy(c_hbm, cv)
        pltpu.sync_copy(w_hbm, nv)
        pltpu.sync_copy(idx_hbm, idxv)
        pltpu.sync_copy(or_hbm, orv)
        pltpu.sync_copy(oc_hbm, ocv)

        for t in range(C // L):
            wv = nv[pl.ds(t * L, L)]
            nv[pl.ds(t * L, L)] = -0.5 / (wv * wv)

        zero = jnp.zeros((L,), jnp.float32)

        def fuzz_body(bp, carry):
            for t in range(C // L):
                col = pl.ds(t * L, L)
                cvt = cv[col]
                nvt = nv[col]
                xe = xs[pl.ds((2 * bp) * C + t * L, L)]
                xo = xs[pl.ds((2 * bp + 1) * C + t * L, L)]
                de = xe - cvt
                do = xo - cvt
                fe = jnp.exp(de * de * nvt)
                fo = jnp.exp(do * do * nvt)
                packed = plsc.pack(fe, fo, format=plsc.PackFormat.INTERLEAVED)
                fz[pl.ds(bp * C + t * L, L)] = plsc.bitcast(packed, jnp.int32)
            for j in range(6):
                acc[pl.ds(bp * 6 * L + j * L, L)] = zero
            return carry

        lax.fori_loop(0, BPT // 2, fuzz_body, 0)

        riota = lax.iota(jnp.int32, L)

        @plsc.parallel_loop(0, NCHUNK, unroll=4)
        def chunk_body(i):
            cvec = riota * A + i * (L * A)
            ia = [plsc.load_gather(idxv, [cvec + a]) for a in range(A)]
            ovec = riota * 2 + i * (L * 2)
            ow0 = plsc.load_gather(ocv, [plsc.load_gather(orv, [ovec])])
            ow1 = plsc.load_gather(ocv, [plsc.load_gather(orv, [ovec + 1])])
            for bp in range(BPT // 2):
                boff = bp * C
                g = [
                    plsc.bitcast(
                        plsc.load_gather(fz, [ia[a] + boff]), jnp.bfloat16
                    )
                    for a in range(A)
                ]
                w01 = jnp.minimum(g[0], g[1])
                w23 = jnp.minimum(g[2], g[3])
                w45 = jnp.minimum(g[4], g[5])
                w = jnp.minimum(jnp.minimum(w01, w23), w45)
                we, wo = plsc.unpack(w, format=plsc.PackFormat.INTERLEAVED)
                o = bp * 6 * L
                plsc.addupdate(acc.at[pl.ds(o, L)], we * ow0)
                plsc.addupdate(acc.at[pl.ds(o + L, L)], we * ow1)
                plsc.addupdate(acc.at[pl.ds(o + 2 * L, L)], we)
                plsc.addupdate(acc.at[pl.ds(o + 3 * L, L)], wo * ow0)
                plsc.addupdate(acc.at[pl.ds(o + 4 * L, L)], wo * ow1)
                plsc.addupdate(acc.at[pl.ds(o + 5 * L, L)], wo)

        pltpu.sync_copy(acc, out_hbm.at[pl.ds(wid * BPT * 3 * L, BPT * 3 * L)])

    return k(x_rep, c_flat, w_flat, idx_t, or_t, oc_pad)


def _tc_rule_kernel(x_rep2, c_flat, w_flat, idx_blk, or_blk, oc_mat):
    nblk = RT // TT

    def body(xr_ref, cv_ref, wv_ref, idx_ref, or_ref, ocm_ref, out_ref,
             fzs, accs):
        step = pl.program_id(0)

        @pl.when(step == 0)
        def _init():
            d = xr_ref[...] - cv_ref[...]
            wv = wv_ref[...]
            fzs[...] = jnp.exp(-(d * d) / (2.0 * wv * wv)).astype(jnp.bfloat16)
            accs[...] = jnp.zeros((B, 3), jnp.float32)

        fzb = fzs[...]
        iota = lax.broadcasted_iota(jnp.int32, (C, TT), 0)
        w = None
        for a in range(A):
            oh = (iota == idx_ref[0, a, :][None, :]).astype(jnp.bfloat16)
            ga = jnp.dot(fzb, oh, preferred_element_type=jnp.float32)
            w = ga if w is None else jnp.minimum(w, ga)
        i16 = lax.broadcasted_iota(jnp.int32, (TT, L), 1)
        ohc = jnp.logical_or(
            i16 == or_ref[0, 0, :][:, None], i16 == or_ref[0, 1, :][:, None]
        ).astype(jnp.float32)
        ow3 = jnp.dot(ohc, ocm_ref[...], preferred_element_type=jnp.float32,
                      precision=jax.lax.Precision.HIGHEST)
        part = jnp.dot(w.astype(jnp.bfloat16), ow3.astype(jnp.bfloat16),
                       preferred_element_type=jnp.float32)
        accs[...] += part

        @pl.when(step == nblk - 1)
        def _fini():
            out_ref[...] = accs[...]

    return pl.pallas_call(
        body,
        grid=(nblk,),
        in_specs=[
            pl.BlockSpec((B, C), lambda j: (0, 0)),
            pl.BlockSpec((1, C), lambda j: (0, 0)),
            pl.BlockSpec((1, C), lambda j: (0, 0)),
            pl.BlockSpec((1, A, TT), lambda j: (j, 0, 0)),
            pl.BlockSpec((1, 2, TT), lambda j: (j, 0, 0)),
            pl.BlockSpec((L, 3), lambda j: (0, 0)),
        ],
        out_specs=pl.BlockSpec((B, 3), lambda j: (0, 0)),
        out_shape=jax.ShapeDtypeStruct((B, 3), jnp.float32),
        scratch_shapes=[
            pltpu.VMEM((B, C), jnp.bfloat16),
            pltpu.VMEM((B, 3), jnp.float32),
        ],
    )(x_rep2, c_flat.reshape(1, C), w_flat.reshape(1, C), idx_blk, or_blk,
      oc_mat)


def _tc_epilogue(acc_flat, tc3, out_scaling, out_bias):
    def body(a_ref, t_ref, s_ref, b_ref, o_ref):
        a = a_ref[...]
        t3 = t_ref[...]
        s0 = jnp.sum(a[:, 0:L], axis=1, keepdims=True) + t3[:, 0:1]
        s1 = jnp.sum(a[:, L:2 * L], axis=1, keepdims=True) + t3[:, 1:2]
        sd = jnp.sum(a[:, 2 * L:3 * L], axis=1, keepdims=True) + t3[:, 2:3]
        denom = jnp.maximum(sd, 1e-12)
        z = jnp.concatenate([s0, s1], axis=1) / denom
        o_ref[...] = jnp.tanh(z) * s_ref[...] + b_ref[...]

    return pl.pallas_call(
        body,
        out_shape=jax.ShapeDtypeStruct((B, 2), jnp.float32),
    )(acc_flat, tc3, out_scaling, out_bias)


def kernel(x, in_centers, in_widths, out_centers, out_scaling, out_bias,
           input_rules, output_rules):
    x_rep2 = jnp.repeat(x, K, axis=1)
    x_rep = x_rep2.reshape(B * C)
    c_flat = in_centers.reshape(C)
    w_flat = in_widths.reshape(C)
    idx_sc = input_rules[:RS].reshape(RS * A)
    or_sc = output_rules[:RS].reshape(RS * 2)
    oc_pad = jnp.pad(out_centers, (0, C - out_centers.shape[0]))

    idx_blk = (input_rules[RS:].reshape(RT // TT, TT, A)
               .transpose(0, 2, 1))
    or_blk = (output_rules[RS:].reshape(RT // TT, TT, 2)
              .transpose(0, 2, 1))
    n12 = out_centers.shape[0]
    ocp = jnp.pad(out_centers, (0, L - n12))
    k16 = jnp.arange(L)
    oc_mat = jnp.stack(
        [jnp.where(k16 < N_ANG, ocp, 0.0),
         jnp.where(k16 >= N_ANG, ocp, 0.0),
         (k16 < N_ANG).astype(jnp.float32)], axis=1)

    acc = _sc_rule_kernel(x_rep, c_flat, w_flat, idx_sc, or_sc, oc_pad)
    tc3 = _tc_rule_kernel(x_rep2, c_flat, w_flat, idx_blk, or_blk, oc_mat)
    return _tc_epilogue(acc.reshape(B, 3 * L), tc3, out_scaling, out_bias)
